# baseline (speedup 1.0000x reference)
import functools

import jax
import jax.numpy as jnp
from jax import lax
from jax.experimental import pallas as pl
from jax.experimental.pallas import tpu as pltpu
from jax.experimental.pallas import tpu_sc as plsc

B = 16384
D = 64
H = 2 * D

_NC = 2
_NS = 16
_NW = _NC * _NS
_BPW = B // _NW
_CH = 256
_LANES = 16


def _sc_gather_both(user_table2, item_table2, user_ids, item_ids):
    mesh = plsc.VectorSubcoreMesh(core_axis_name="c", subcore_axis_name="s")

    @functools.partial(
        pl.kernel,
        mesh=mesh,
        out_type=(
            jax.ShapeDtypeStruct((B, 2 * D), jnp.float32),
            jax.ShapeDtypeStruct((B, 2 * D), jnp.float32),
        ),
        scratch_types=[
            pltpu.VMEM((_BPW,), jnp.int32),
            pltpu.VMEM((_BPW,), jnp.int32),
            pltpu.VMEM((_CH, 2 * D), jnp.float32),
            pltpu.VMEM((_CH, 2 * D), jnp.float32),
            pltpu.SemaphoreType.DMA,
            pltpu.SemaphoreType.DMA,
            pltpu.SemaphoreType.DMA,
            pltpu.SemaphoreType.DMA,
        ],
    )
    def k(ut_hbm, it_hbm, uid_hbm, iid_hbm, uout_hbm, iout_hbm,
          uidx_v, iidx_v, rows_a, rows_b, sem_a, sem_b, sem_sa, sem_sb):
        wid = lax.axis_index("s") * _NC + lax.axis_index("c")
        base = wid * _BPW
        sl = pl.ds(base, _BPW)
        pltpu.sync_copy(uid_hbm.at[sl], uidx_v)
        pltpu.sync_copy(iid_hbm.at[sl], iidx_v)

        @pl.loop(0, _BPW, step=_LANES)
        def _(c):
            s = pl.ds(c, _LANES)
            uidx_v.at[s][...] = lax.shift_right_logical(uidx_v.at[s][...], 1)
            iidx_v.at[s][...] = lax.shift_right_logical(iidx_v.at[s][...], 1)

        work = []
        for tbl, idx_v, out_hbm in ((ut_hbm, uidx_v, uout_hbm),
                                    (it_hbm, iidx_v, iout_hbm)):
            for c in range(_BPW // _CH):
                work.append((tbl, idx_v.at[pl.ds(c * _CH, _CH)],
                             out_hbm.at[pl.ds(base + c * _CH, _CH)]))

        bufs = (rows_a, rows_b)
        gsems = (sem_a, sem_b)
        ssems = (sem_sa, sem_sb)
        n = len(work)
        gath = [None] * n
        stor = [None] * n
        for i in range(n):
            b = i % 2
            if i >= 2:
                stor[i - 2].wait()
            tbl, idx, out = work[i]
            gath[i] = pltpu.async_copy(tbl.at[idx], bufs[b], gsems[b])
            if i >= 1:
                gath[i - 1].wait()
                _, _, prev_out = work[i - 1]
                stor[i - 1] = pltpu.async_copy(bufs[(i - 1) % 2], prev_out,
                                               ssems[(i - 1) % 2])
        gath[n - 1].wait()
        stor[n - 1] = pltpu.async_copy(bufs[(n - 1) % 2], work[n - 1][2],
                                       ssems[(n - 1) % 2])
        stor[n - 2].wait()
        stor[n - 1].wait()

    return k(user_table2, item_table2, user_ids, item_ids)

    return k(user_table2, item_table2, user_ids, item_ids)


_BLK = 2048


def _mlp_body(eu_ref, ei_ref, uids_ref, iids_ref,
              uw1, ub1, uw2, ub2, iw1, ib1, iw2, ib2,
              ou_ref, oi_ref):
    def tower(e2, ids, w1, b1, w2, b2):
        odd = jnp.bitwise_and(ids, 1) == 1
        e = jnp.where(odd, e2[:, D:], e2[:, :D])
        h = jnp.dot(e, w1, preferred_element_type=jnp.float32,
                    precision=lax.Precision.HIGHEST)
        h = jnp.maximum(h + b1, 0.0)
        o = jnp.dot(h, w2, preferred_element_type=jnp.float32,
                    precision=lax.Precision.HIGHEST)
        o = o + b2
        norm = jnp.sqrt(jnp.sum(o * o, axis=1, keepdims=True))
        return o / jnp.maximum(norm, 1e-12)

    ou_ref[...] = tower(eu_ref[...], uids_ref[...],
                        uw1[...], ub1[...], uw2[...], ub2[...])
    oi_ref[...] = tower(ei_ref[...], iids_ref[...],
                        iw1[...], ib1[...], iw2[...], ib2[...])


def _mlp_norm(e2_u, e2_i, uids_r, iids_r, uW1, ub1, uW2, ub2, iW1, ib1, iW2, ib2):
    blk2 = pl.BlockSpec((_BLK, 2 * D), lambda i: (i, 0))
    blk = pl.BlockSpec((_BLK, D), lambda i: (i, 0))
    ids_spec = pl.BlockSpec((_BLK, 1), lambda i: (i, 0))
    full = lambda shape: pl.BlockSpec(shape, lambda i: tuple(0 for _ in shape))
    return pl.pallas_call(
        _mlp_body,
        grid=(B // _BLK,),
        in_specs=[
            blk2, blk2, ids_spec, ids_spec,
            full((D, H)), full((1, H)), full((H, D)), full((1, D)),
            full((D, H)), full((1, H)), full((H, D)), full((1, D)),
        ],
        out_specs=[blk, blk],
        out_shape=(
            jax.ShapeDtypeStruct((B, D), jnp.float32),
            jax.ShapeDtypeStruct((B, D), jnp.float32),
        ),
    )(e2_u, e2_i, uids_r, iids_r, uW1, ub1, uW2, ub2, iW1, ib1, iW2, ib2)


def kernel(user_ids, item_ids, user_table, item_table,
           uW1, ub1, uW2, ub2, iW1, ib1, iW2, ib2):
    ut2 = user_table.reshape(-1, 2 * D)
    it2 = item_table.reshape(-1, 2 * D)
    e2_u, e2_i = _sc_gather_both(ut2, it2, user_ids, item_ids)
    u_vec, i_vec = _mlp_norm(
        e2_u, e2_i,
        user_ids.reshape(B, 1), item_ids.reshape(B, 1),
        uW1, ub1.reshape(1, H), uW2, ub2.reshape(1, D),
        iW1, ib1.reshape(1, H), iW2, ib2.reshape(1, D),
    )
    return (u_vec, i_vec)
